# baseline (speedup 1.0000x reference)
import functools

import jax
import jax.numpy as jnp
from jax import lax
from jax.experimental import pallas as pl
from jax.experimental.pallas import tpu as pltpu
from jax.experimental.pallas import tpu_sc as plsc

N_NODES = 100000
F_X = 128
N_GRAPHS = 64
F_OUT = 128

NC = 2
NS = 16
NW = NC * NS

CHUNK = 128
N_CHUNKS = N_NODES // CHUNK
N_MAIN = N_CHUNKS * CHUNK
TAIL = N_NODES - N_MAIN


STEPS = -(-N_CHUNKS // NW)
N_FULL_W = N_CHUNKS - (STEPS - 1) * NW
NBUF = 4


def _sc_segment_partials(x, batch):
    mesh = plsc.VectorSubcoreMesh(
        core_axis_name="c", subcore_axis_name="s", num_cores=NC, num_subcores=NS
    )

    @functools.partial(
        pl.kernel,
        out_type=jax.ShapeDtypeStruct((NW, N_GRAPHS, F_X), jnp.float32),
        mesh=mesh,
        scratch_types=[
            pltpu.VMEM((NBUF, CHUNK, F_X), jnp.float32),
            pltpu.VMEM((NBUF, CHUNK), jnp.int32),
            pltpu.VMEM((N_GRAPHS, F_X), jnp.float32),
            pltpu.VMEM((TAIL, F_X), jnp.float32),
            pltpu.VMEM((TAIL,), jnp.int32),
            [pltpu.SemaphoreType.DMA] * NBUF,
        ],
    )
    def sc_kernel(x_hbm, b_hbm, out_hbm, rows_v, idx_v, acc_v, trow_v, tidx_v,
                  lsem):
        cid = lax.axis_index("c")
        sid = lax.axis_index("s")
        wid = sid * NC + cid
        n_mine = STEPS - 1 + (wid < N_FULL_W).astype(jnp.int32)

        def issue_load(i, b):
            base = (wid + i * NW) * CHUNK
            pltpu.async_copy(b_hbm.at[pl.ds(base, CHUNK)], idx_v.at[b],
                             lsem[b])
            pltpu.async_copy(x_hbm.at[pl.ds(base, CHUNK)], rows_v.at[b],
                             lsem[b])

        def wait_load(i, b):
            base = (wid + i * NW) * CHUNK
            pltpu.make_async_copy(b_hbm.at[pl.ds(base, CHUNK)],
                                  idx_v.at[b], lsem[b]).wait()
            pltpu.make_async_copy(x_hbm.at[pl.ds(base, CHUNK)],
                                  rows_v.at[b], lsem[b]).wait()

        def accumulate(buf):
            def grp_body(k, carry):
                r0 = k * 16
                gvec = idx_v[buf, pl.ds(r0, 16)]
                for j in range(16):
                    g = gvec[j]
                    for f in range(F_X // 16):
                        plsc.addupdate(
                            acc_v.at[g, pl.ds(16 * f, 16)],
                            rows_v[buf, r0 + j, pl.ds(16 * f, 16)],
                        )
                return carry

            lax.fori_loop(0, CHUNK // 16, grp_body, 0)

        for i in range(NBUF - 1):
            issue_load(i, i)

        zeros = jnp.zeros((16,), jnp.float32)

        def zero_row(g, carry):
            for f in range(F_X // 16):
                acc_v[g, pl.ds(16 * f, 16)] = zeros
            return carry

        lax.fori_loop(0, N_GRAPHS, zero_row, 0)

        def group_body(gi, carry):
            for b in range(NBUF):
                i = gi * NBUF + b
                pf = i + NBUF - 1

                @pl.when(pf < n_mine)
                def _(pf=pf, b=b):
                    issue_load(pf, (b + NBUF - 1) % NBUF)

                @pl.when(i < n_mine)
                def _(i=i, b=b):
                    wait_load(i, b)
                    accumulate(b)
            return carry

        lax.fori_loop(0, -(-STEPS // NBUF), group_body, 0)

        @pl.when(wid == 0)
        def _tail():
            pltpu.sync_copy(b_hbm.at[pl.ds(N_MAIN, TAIL)], tidx_v)
            pltpu.sync_copy(x_hbm.at[pl.ds(N_MAIN, TAIL)], trow_v)

            for k in range(TAIL // 16):
                gvec = tidx_v[pl.ds(k * 16, 16)]
                for j in range(16):
                    g = gvec[j]
                    for f in range(F_X // 16):
                        plsc.addupdate(
                            acc_v.at[g, pl.ds(16 * f, 16)],
                            trow_v[k * 16 + j, pl.ds(16 * f, 16)],
                        )

        pltpu.sync_copy(acc_v, out_hbm.at[wid])

    return sc_kernel(x, batch)


def _tc_head(partials, u, W, b2d):
    def tc_kernel(p_ref, u_ref, w_ref, b_ref, o_ref):
        x_agg = jnp.sum(p_ref[...], axis=0)
        out = jnp.dot(x_agg, w_ref[:F_X, :], preferred_element_type=jnp.float32)
        out = out + jnp.dot(u_ref[...], w_ref[F_X:, :], preferred_element_type=jnp.float32)
        o_ref[...] = out + b_ref[...]

    return pl.pallas_call(
        tc_kernel,
        out_shape=jax.ShapeDtypeStruct((N_GRAPHS, F_OUT), jnp.float32),
    )(partials, u, W, b2d)


def kernel(x, edge_index, e, u, batch, W, b):
    del edge_index, e
    batch32 = batch.astype(jnp.int32)
    partials = _sc_segment_partials(x, batch32)
    return _tc_head(partials, u, W, b.reshape(1, F_OUT).astype(jnp.float32))


# speedup vs baseline: 2.1388x; 2.1388x over previous
import functools

import jax
import jax.numpy as jnp
from jax import lax
from jax.experimental import pallas as pl
from jax.experimental.pallas import tpu as pltpu
from jax.experimental.pallas import tpu_sc as plsc

N_NODES = 100000
F_X = 128
N_GRAPHS = 64
F_OUT = 128

NC = 2
NS = 16
NW = NC * NS

CHUNK = 128

N_SC = 48000
TC_BLK = 1000
N_TC_BLKS = (N_NODES - N_SC) // TC_BLK

N_CHUNKS = N_SC // CHUNK
STEPS = -(-N_CHUNKS // NW)
N_FULL_W = N_CHUNKS - (STEPS - 1) * NW
NBUF = 4


def _sc_segment_partials(x, batch):
    mesh = plsc.VectorSubcoreMesh(
        core_axis_name="c", subcore_axis_name="s", num_cores=NC, num_subcores=NS
    )
    zrows = N_GRAPHS // NS

    @functools.partial(
        pl.kernel,
        out_type=jax.ShapeDtypeStruct((NC, N_GRAPHS, F_X), jnp.float32),
        mesh=mesh,
        scratch_types=[
            pltpu.VMEM((NBUF, CHUNK, F_X), jnp.float32),
            pltpu.VMEM((NBUF, CHUNK), jnp.int32),
            pltpu.VMEM((zrows, F_X), jnp.float32),
            pltpu.VMEM_SHARED((N_GRAPHS, F_X), jnp.float32),
            [pltpu.SemaphoreType.DMA] * NBUF,
            [pltpu.SemaphoreType.DMA] * NBUF,
        ],
    )
    def sc_kernel(x_hbm, b_hbm, out_hbm, rows_v, idx_v, zbuf_v,
                  acc_sh, lsem, ssem):
        cid = lax.axis_index("c")
        sid = lax.axis_index("s")
        wid = sid * NC + cid
        valid_last = wid < N_FULL_W

        def maybe(i, fn):
            if i == STEPS - 1:
                pl.when(valid_last)(fn)
            else:
                fn()

        def issue_load(i):
            base = (wid + i * NW) * CHUNK
            pltpu.async_copy(b_hbm.at[pl.ds(base, CHUNK)], idx_v.at[i % NBUF],
                             lsem[i % NBUF])
            pltpu.async_copy(x_hbm.at[pl.ds(base, CHUNK)], rows_v.at[i % NBUF],
                             lsem[i % NBUF])

        def wait_load(i):
            base = (wid + i * NW) * CHUNK
            pltpu.make_async_copy(b_hbm.at[pl.ds(base, CHUNK)],
                                  idx_v.at[i % NBUF], lsem[i % NBUF]).wait()
            pltpu.make_async_copy(x_hbm.at[pl.ds(base, CHUNK)],
                                  rows_v.at[i % NBUF], lsem[i % NBUF]).wait()

        def issue_scatter(i):
            pltpu.async_copy(rows_v.at[i % NBUF], acc_sh.at[idx_v.at[i % NBUF]],
                             ssem[i % NBUF], priority=1, add=True)

        def wait_scatter(i):
            pltpu.make_async_copy(rows_v.at[i % NBUF],
                                  acc_sh.at[idx_v.at[i % NBUF]],
                                  ssem[i % NBUF]).wait()

        for i in range(NBUF - 1):
            issue_load(i)

        zeros = jnp.zeros((16,), jnp.float32)
        for r in range(zrows):
            for f in range(F_X // 16):
                zbuf_v[r, pl.ds(16 * f, 16)] = zeros
        pltpu.sync_copy(zbuf_v, acc_sh.at[pl.ds(sid * zrows, zrows)])
        plsc.subcore_barrier()

        for i in range(STEPS):
            def step(i=i):
                wait_load(i)
                issue_scatter(i)
            maybe(i, step)
            f = i + NBUF - 1
            if f < STEPS:
                def prefetch(f=f):
                    if f >= NBUF:
                        wait_scatter(f - NBUF)
                    issue_load(f)
                maybe(f, prefetch)

        for i in range(STEPS - NBUF, STEPS):
            maybe(i, lambda i=i: wait_scatter(i))

        plsc.subcore_barrier()

        @pl.when(sid == 0)
        def _writeback():
            pltpu.sync_copy(acc_sh, out_hbm.at[cid])

    return sc_kernel(x, batch)


def _tc_segment_partial(x, batch3d):
    def tc_kernel(b_ref, x_ref, o_ref, acc_ref):
        i = pl.program_id(0)

        @pl.when(i == 0)
        def _():
            acc_ref[...] = jnp.zeros((N_GRAPHS, F_X), jnp.float32)

        ids = b_ref[0]
        gids = lax.broadcasted_iota(jnp.int32, (N_GRAPHS, TC_BLK), 0)
        one_hot = jnp.where(gids == ids, 1.0, 0.0).astype(jnp.float32)
        acc_ref[...] += jnp.dot(one_hot, x_ref[...],
                                preferred_element_type=jnp.float32)

        @pl.when(i == N_TC_BLKS - 1)
        def _():
            o_ref[...] = acc_ref[...]

    off = N_SC // TC_BLK
    return pl.pallas_call(
        tc_kernel,
        grid=(N_TC_BLKS,),
        in_specs=[
            pl.BlockSpec((1, 1, TC_BLK), lambda i: (i + off, 0, 0)),
            pl.BlockSpec((TC_BLK, F_X), lambda i: (i + off, 0)),
        ],
        out_specs=pl.BlockSpec((N_GRAPHS, F_X), lambda i: (0, 0)),
        out_shape=jax.ShapeDtypeStruct((N_GRAPHS, F_X), jnp.float32),
        scratch_shapes=[pltpu.VMEM((N_GRAPHS, F_X), jnp.float32)],
    )(batch3d, x)


def _tc_head(sc_partials, tc_partial, u, W, b2d):
    def tc_kernel(p_ref, t_ref, u_ref, w_ref, b_ref, o_ref):
        x_agg = p_ref[0] + p_ref[1] + t_ref[...]
        out = jnp.dot(x_agg, w_ref[:F_X, :], preferred_element_type=jnp.float32)
        out = out + jnp.dot(u_ref[...], w_ref[F_X:, :], preferred_element_type=jnp.float32)
        o_ref[...] = out + b_ref[...]

    return pl.pallas_call(
        tc_kernel,
        out_shape=jax.ShapeDtypeStruct((N_GRAPHS, F_OUT), jnp.float32),
    )(sc_partials, tc_partial, u, W, b2d)


def kernel(x, edge_index, e, u, batch, W, b):
    del edge_index, e
    batch32 = batch.astype(jnp.int32)
    batch3d = batch32.reshape(N_NODES // TC_BLK, 1, TC_BLK)
    sc_partials = _sc_segment_partials(x, batch32)
    tc_partial = _tc_segment_partial(x, batch3d)
    return _tc_head(sc_partials, tc_partial, u, W,
                    b.reshape(1, F_OUT).astype(jnp.float32))


# speedup vs baseline: 2.3603x; 1.1036x over previous
import functools

import jax
import jax.numpy as jnp
from jax import lax
from jax.experimental import pallas as pl
from jax.experimental.pallas import tpu as pltpu
from jax.experimental.pallas import tpu_sc as plsc

N_NODES = 100000
F_X = 128
N_GRAPHS = 64
F_OUT = 128

NC = 2
NS = 16
NW = NC * NS

CHUNK = 128
N_CHUNKS = N_NODES // CHUNK
N_MAIN = N_CHUNKS * CHUNK
TAIL = N_NODES - N_MAIN


STEPS = -(-N_CHUNKS // NW)
N_FULL_W = N_CHUNKS - (STEPS - 1) * NW
NBUF = 6


def _sc_segment_partials(x, batch):
    mesh = plsc.VectorSubcoreMesh(
        core_axis_name="c", subcore_axis_name="s", num_cores=NC, num_subcores=NS
    )
    zrows = N_GRAPHS // NS

    @functools.partial(
        pl.kernel,
        out_type=jax.ShapeDtypeStruct((NC, N_GRAPHS, F_X), jnp.float32),
        mesh=mesh,
        scratch_types=[
            pltpu.VMEM((NBUF, CHUNK, F_X), jnp.float32),
            pltpu.VMEM((NBUF, CHUNK), jnp.int32),
            pltpu.VMEM((zrows, F_X), jnp.float32),
            pltpu.VMEM((TAIL, F_X), jnp.float32),
            pltpu.VMEM((TAIL,), jnp.int32),
            pltpu.VMEM_SHARED((N_GRAPHS, F_X), jnp.float32),
            [pltpu.SemaphoreType.DMA] * NBUF,
            [pltpu.SemaphoreType.DMA] * NBUF,
        ],
    )
    def sc_kernel(x_hbm, b_hbm, out_hbm, rows_v, idx_v, zbuf_v, trow_v, tidx_v,
                  acc_sh, lsem, ssem):
        cid = lax.axis_index("c")
        sid = lax.axis_index("s")
        wid = sid * NC + cid
        valid_last = wid < N_FULL_W

        def maybe(i, fn):
            if i == STEPS - 1:
                pl.when(valid_last)(fn)
            else:
                fn()

        def issue_load(i):
            base = (wid + i * NW) * CHUNK
            pltpu.async_copy(b_hbm.at[pl.ds(base, CHUNK)], idx_v.at[i % NBUF],
                             lsem[i % NBUF])
            pltpu.async_copy(x_hbm.at[pl.ds(base, CHUNK)], rows_v.at[i % NBUF],
                             lsem[i % NBUF])

        def wait_load(i):
            base = (wid + i * NW) * CHUNK
            pltpu.make_async_copy(b_hbm.at[pl.ds(base, CHUNK)],
                                  idx_v.at[i % NBUF], lsem[i % NBUF]).wait()
            pltpu.make_async_copy(x_hbm.at[pl.ds(base, CHUNK)],
                                  rows_v.at[i % NBUF], lsem[i % NBUF]).wait()

        def issue_scatter(i):
            pltpu.async_copy(rows_v.at[i % NBUF], acc_sh.at[idx_v.at[i % NBUF]],
                             ssem[i % NBUF], priority=1, add=True)

        def wait_scatter(i):
            pltpu.make_async_copy(rows_v.at[i % NBUF],
                                  acc_sh.at[idx_v.at[i % NBUF]],
                                  ssem[i % NBUF]).wait()

        for i in range(NBUF - 1):
            issue_load(i)

        zeros = jnp.zeros((16,), jnp.float32)
        for r in range(zrows):
            for f in range(F_X // 16):
                zbuf_v[r, pl.ds(16 * f, 16)] = zeros
        pltpu.sync_copy(zbuf_v, acc_sh.at[pl.ds(sid * zrows, zrows)])
        plsc.subcore_barrier()

        for i in range(STEPS):
            def step(i=i):
                wait_load(i)
                issue_scatter(i)
            maybe(i, step)
            f = i + NBUF - 1
            if f < STEPS:
                def prefetch(f=f):
                    if f >= NBUF:
                        wait_scatter(f - NBUF)
                    issue_load(f)
                maybe(f, prefetch)

        for i in range(STEPS - NBUF, STEPS):
            maybe(i, lambda i=i: wait_scatter(i))

        @pl.when(wid == 0)
        def _tail():
            pltpu.sync_copy(b_hbm.at[pl.ds(N_MAIN, TAIL)], tidx_v)
            pltpu.sync_copy(x_hbm.at[pl.ds(N_MAIN, TAIL)], trow_v)
            pltpu.sync_copy(trow_v, acc_sh.at[tidx_v], add=True)

        plsc.subcore_barrier()

        @pl.when(sid == 0)
        def _writeback():
            pltpu.sync_copy(acc_sh, out_hbm.at[cid])

    return sc_kernel(x, batch)


def _tc_head(partials, u, W, b2d):
    def tc_kernel(p_ref, u_ref, w_ref, b_ref, o_ref):
        x_agg = p_ref[0] + p_ref[1]
        out = jnp.dot(x_agg, w_ref[:F_X, :], preferred_element_type=jnp.float32)
        out = out + jnp.dot(u_ref[...], w_ref[F_X:, :], preferred_element_type=jnp.float32)
        o_ref[...] = out + b_ref[...]

    return pl.pallas_call(
        tc_kernel,
        out_shape=jax.ShapeDtypeStruct((N_GRAPHS, F_OUT), jnp.float32),
    )(partials, u, W, b2d)


def kernel(x, edge_index, e, u, batch, W, b):
    del edge_index, e
    batch32 = batch.astype(jnp.int32)
    partials = _sc_segment_partials(x, batch32)
    return _tc_head(partials, u, W, b.reshape(1, F_OUT).astype(jnp.float32))
